# baseline (speedup 1.0000x reference)
import functools

import jax
import jax.numpy as jnp
from jax import lax
from jax.experimental import pallas as pl
from jax.experimental.pallas import tpu as pltpu
from jax.experimental.pallas import tpu_sc as plsc

N = 10000
D = 128
L = 3
E = 320000

NC = 2
NS = 16
FT = NC * NS
NH = N // 2
LH = 5120
TRASH_H = NH
SLICE = 10240
E_PAD = FT * SLICE
G = 4
CAP = 12288
BLK = 2048
NBLK = CAP // BLK
CH = 64
SROW = 320
BR = 1000

_mesh = plsc.VectorSubcoreMesh(core_axis_name="c", subcore_axis_name="s")
_sc_params = pltpu.CompilerParams(needs_layout_passes=False)


def _filter_body(src_hbm, dst_hbm, lsrc_hbm, ldst_hbm, counts_hbm,
                 sv, dv, ob_src, ob_dst, cntv):
    cid = lax.axis_index("c")
    sid = lax.axis_index("s")
    wid = cid * NS + sid
    pltpu.sync_copy(src_hbm.at[pl.ds(wid * SLICE, SLICE)], sv)
    pltpu.sync_copy(dst_hbm.at[pl.ds(wid * SLICE, SLICE)], dv)

    zv = jnp.zeros((16,), jnp.int32)
    iota = lax.iota(jnp.int32, 16)

    def step(k, offs):
        vs = sv[pl.ds(k * 16, 16)]
        vd = dv[pl.ds(k * 16, 16)]
        sa = vs >= NH
        sb = vd >= NH
        vsl = [vs, vs, vs - NH, vs - NH]
        vdl = [vd, vd - NH, vd, vd - NH]
        masks = [(~sa) & (~sb), (~sa) & sb, sa & (~sb), sa & sb]
        new = []
        for g in range(G):
            ov = offs[g]
            mi = masks[g].astype(jnp.int32)
            pos = plsc.cumsum(mi)
            idx = (g * CAP - 1) + ov + pos
            plsc.store_scatter(ob_src, [idx], vsl[g], mask=masks[g])
            plsc.store_scatter(ob_dst, [idx], vdl[g], mask=masks[g])
            new.append(ov + plsc.all_reduce_population_count(masks[g]))
        return tuple(new)

    offs = lax.fori_loop(0, SLICE // 16, step, (zv, zv, zv, zv))

    zpad = jnp.zeros((16,), jnp.int32)
    tpad = jnp.full((16,), TRASH_H, jnp.int32)
    for g in range(G):
        ov = offs[g]
        for t in range(4):
            idx = (g * CAP + t * 16) + ov + iota
            plsc.store_scatter(ob_src, [idx], zpad)
            plsc.store_scatter(ob_dst, [idx], tpad)
        pv = ((ov + 63) >> 6) << 6
        cntv[pl.ds(g * 16, 16)] = pv
    pltpu.sync_copy(cntv, counts_hbm.at[wid])
    pltpu.sync_copy(ob_src, lsrc_hbm.at[wid])
    pltpu.sync_copy(ob_dst, ldst_hbm.at[wid])


_filter = pl.kernel(
    _filter_body,
    out_type=(
        jax.ShapeDtypeStruct((FT, G * CAP), jnp.int32),
        jax.ShapeDtypeStruct((FT, G * CAP), jnp.int32),
        jax.ShapeDtypeStruct((FT, G * 16), jnp.int32),
    ),
    mesh=_mesh,
    compiler_params=_sc_params,
    scratch_types=[
        pltpu.VMEM((SLICE,), jnp.int32),
        pltpu.VMEM((SLICE,), jnp.int32),
        pltpu.VMEM((G * CAP,), jnp.int32),
        pltpu.VMEM((G * CAP,), jnp.int32),
        pltpu.VMEM((G * 16,), jnp.int32),
    ],
)


def _make_pass(pass_idx):
    def body(m_hbm, lsrc, ldst, counts_hbm, *rest):
        if pass_idx == 0:
            (out_hbm, m_s, agg_s, counts_v, sv, dv, r0, r1, s0, s1) = rest
        else:
            (part_in, out_hbm, m_s, agg_s, counts_v,
             sv, dv, r0, r1, s0, s1) = rest
        cid = lax.axis_index("c")
        sid = lax.axis_index("s")
        a = cid
        b = cid if pass_idx == 0 else 1 - cid
        g = a * 2 + b
        rows = [r0, r1]
        ss = [s0, s1]

        pl.when(sid < 15)(lambda: pltpu.sync_copy(
            m_hbm.at[pl.ds(a * NH + sid * SROW, SROW)],
            m_s.at[pl.ds(sid * SROW, SROW)]))
        pl.when(sid == 15)(lambda: pltpu.sync_copy(
            m_hbm.at[pl.ds(a * NH + 15 * SROW, NH - 15 * SROW)],
            m_s.at[pl.ds(15 * SROW, NH - 15 * SROW)]))

        if pass_idx == 0:
            zv = jnp.zeros((16,), jnp.float32)

            def _zrow(r, carry):
                for jj in range(8):
                    r0[r, pl.ds(jj * 16, 16)] = zv
                return carry

            lax.fori_loop(0, CH, _zrow, 0)
            for k in range(SROW // CH):
                pltpu.sync_copy(
                    r0, agg_s.at[pl.ds(sid * SROW + k * CH, CH)])
        else:
            pltpu.sync_copy(part_in.at[b, pl.ds(sid * SROW, SROW)],
                            agg_s.at[pl.ds(sid * SROW, SROW)])
        pltpu.sync_copy(counts_hbm, counts_v)
        plsc.subcore_barrier()

        def sstart(s, bb):
            pltpu.async_copy(rows[bb], agg_s.at[dv.at[s]], ss[bb], add=True)

        def swait(s, bb):
            pltpu.make_async_copy(rows[bb], agg_s.at[dv.at[s]],
                                  ss[bb]).wait()

        for j in range(2):
            ft = sid * 2 + j
            n = counts_v[ft, pl.ds(g * 16, 16)][0]
            nch = n >> 6
            nb = (nch + 31) >> 5

            def blk_body(bi, carry, ft=ft):
                pltpu.sync_copy(lsrc.at[ft, g, bi], sv)
                pltpu.sync_copy(ldst.at[ft, g, bi], dv)
                base = bi * 32
                for s in range(32):
                    bb = s % 2

                    def do(s=s, bb=bb):
                        if s >= 2:
                            swait(s - 2, bb)
                        pltpu.sync_copy(
                            m_s.at[sv.at[pl.ds(s * CH, CH)]], rows[bb])
                        sstart(s, bb)

                    pl.when(base + s < nch)(do)
                for s in range(32):
                    started = base + s < nch
                    last2 = (s >= 30) | (base + s >= nch - 2)
                    pl.when(started & last2)(lambda s=s: swait(s, s % 2))
                return carry

            lax.fori_loop(0, nb, blk_body, 0)

        plsc.subcore_barrier()
        pltpu.sync_copy(agg_s.at[pl.ds(sid * SROW, SROW)],
                        out_hbm.at[b, pl.ds(sid * SROW, SROW)])

    return body


def _pass_scratch():
    return [
        pltpu.VMEM_SHARED((LH, D), jnp.float32),
        pltpu.VMEM_SHARED((LH, D), jnp.float32),
        pltpu.VMEM((FT, G * 16), jnp.int32),
        pltpu.VMEM((BLK,), jnp.int32),
        pltpu.VMEM((32, CH), jnp.int32),
        pltpu.VMEM((CH, D), jnp.float32),
        pltpu.VMEM((CH, D), jnp.float32),
        pltpu.SemaphoreType.DMA,
        pltpu.SemaphoreType.DMA,
    ]


_pass0 = pl.kernel(
    _make_pass(0),
    out_type=jax.ShapeDtypeStruct((NC, LH, D), jnp.float32),
    mesh=_mesh,
    compiler_params=_sc_params,
    scratch_types=_pass_scratch(),
)

_pass1 = pl.kernel(
    _make_pass(1),
    out_type=jax.ShapeDtypeStruct((NC, LH, D), jnp.float32),
    mesh=_mesh,
    compiler_params=_sc_params,
    scratch_types=_pass_scratch(),
)


def _dot(a, b):
    return jnp.dot(a, b, precision=lax.Precision.HIGHEST,
                   preferred_element_type=jnp.float32)


def _lin_in_body(x_ref, wT_ref, b_ref, cw_ref, h_ref, m_ref):
    h = _dot(x_ref[...], wT_ref[...]) + b_ref[...]
    h_ref[...] = h
    m_ref[...] = _dot(h, cw_ref[...])


_lin_in = pl.pallas_call(
    _lin_in_body,
    grid=(N // BR,),
    in_specs=[
        pl.BlockSpec((BR, D), lambda i: (i, 0)),
        pl.BlockSpec((D, D), lambda i: (0, 0)),
        pl.BlockSpec((1, D), lambda i: (0, 0)),
        pl.BlockSpec((D, D), lambda i: (0, 0)),
    ],
    out_specs=[
        pl.BlockSpec((BR, D), lambda i: (i, 0)),
        pl.BlockSpec((BR, D), lambda i: (i, 0)),
    ],
    out_shape=[jax.ShapeDtypeStruct((N, D), jnp.float32)] * 2,
)


def _gru_body(agg_ref, h_ref, wihT_ref, whhT_ref, bih_ref, bhh_ref,
              nw_ref, nb_ref, ho_ref, y_ref):
    agg = agg_ref[0]
    h = h_ref[...]
    gi = _dot(agg, wihT_ref[...]) + bih_ref[...]
    gh = _dot(h, whhT_ref[...]) + bhh_ref[...]
    r = jax.nn.sigmoid(gi[:, :D] + gh[:, :D])
    z = jax.nn.sigmoid(gi[:, D:2 * D] + gh[:, D:2 * D])
    n = jnp.tanh(gi[:, 2 * D:] + r * gh[:, 2 * D:])
    hn = (1.0 - z) * n + z * h
    ho_ref[...] = hn
    y_ref[...] = _dot(hn, nw_ref[...]) + nb_ref[...]


_gru = pl.pallas_call(
    _gru_body,
    grid=(N // BR,),
    in_specs=[
        pl.BlockSpec((1, BR, D), lambda i: (i // 5, i % 5, 0)),
        pl.BlockSpec((BR, D), lambda i: (i, 0)),
        pl.BlockSpec((D, 3 * D), lambda i: (0, 0)),
        pl.BlockSpec((D, 3 * D), lambda i: (0, 0)),
        pl.BlockSpec((1, 3 * D), lambda i: (0, 0)),
        pl.BlockSpec((1, 3 * D), lambda i: (0, 0)),
        pl.BlockSpec((D, D), lambda i: (0, 0)),
        pl.BlockSpec((1, D), lambda i: (0, 0)),
    ],
    out_specs=[
        pl.BlockSpec((BR, D), lambda i: (i, 0)),
        pl.BlockSpec((BR, D), lambda i: (i, 0)),
    ],
    out_shape=[jax.ShapeDtypeStruct((N, D), jnp.float32)] * 2,
)


def kernel(x, edge_index, W_in, b_in, conv_w, gru_w_ih, gru_w_hh,
           gru_b_ih, gru_b_hh, W_out, b_out):
    src = edge_index[0]
    dst = edge_index[1]
    pad = E_PAD - E
    src_p = jnp.concatenate([src, jnp.zeros((pad,), src.dtype)])
    dst_p = jnp.concatenate([dst, jnp.full((pad,), N, dst.dtype)])

    lsrc, ldst, counts = _filter(src_p, dst_p)
    lsrc = lsrc.reshape(FT, G, NBLK, BLK)
    ldst = ldst.reshape(FT, G, NBLK, 32, CH)

    h, m = _lin_in(x, W_in.T, b_in.reshape(1, D), conv_w[0])

    w_ihT = gru_w_ih.T
    w_hhT = gru_w_hh.T
    b_ih2 = gru_b_ih.reshape(1, 3 * D)
    b_hh2 = gru_b_hh.reshape(1, 3 * D)
    zero_b = jnp.zeros((1, D), jnp.float32)
    nexts = [(conv_w[1], zero_b), (conv_w[2], zero_b),
             (W_out.T, b_out.reshape(1, D))]
    for i in range(L):
        part = _pass0(m, lsrc, ldst, counts)
        agg = _pass1(m, lsrc, ldst, counts, part)
        h, m = _gru(agg, h, w_ihT, w_hhT, b_ih2, b_hh2,
                    nexts[i][0], nexts[i][1])
    return m


# speedup vs baseline: 1.0010x; 1.0010x over previous
import functools

import jax
import jax.numpy as jnp
from jax import lax
from jax.experimental import pallas as pl
from jax.experimental.pallas import tpu as pltpu
from jax.experimental.pallas import tpu_sc as plsc

N = 10000
D = 128
L = 3
E = 320000

NC = 2
NS = 16
FT = NC * NS
NH = N // 2
LH = 5120
TRASH_H = NH
SLICE = 10240
E_PAD = FT * SLICE
G = 4
CAP = 12288
BLK = 2048
NBLK = CAP // BLK
CH = 64
SROW = 320
BR = 1000

_mesh = plsc.VectorSubcoreMesh(core_axis_name="c", subcore_axis_name="s")
_sc_params = pltpu.CompilerParams(needs_layout_passes=False)


def _filter_body(src_hbm, dst_hbm, lsrc_hbm, ldst_hbm, counts_hbm,
                 sv, dv, ob_src, ob_dst, cntv):
    cid = lax.axis_index("c")
    sid = lax.axis_index("s")
    wid = cid * NS + sid
    pltpu.sync_copy(src_hbm.at[pl.ds(wid * SLICE, SLICE)], sv)
    pltpu.sync_copy(dst_hbm.at[pl.ds(wid * SLICE, SLICE)], dv)

    zv = jnp.zeros((16,), jnp.int32)
    iota = lax.iota(jnp.int32, 16)

    def step(k, offs):
        vs = sv[pl.ds(k * 16, 16)]
        vd = dv[pl.ds(k * 16, 16)]
        sa = vs >= NH
        sb = vd >= NH
        vsl = [vs, vs, vs - NH, vs - NH]
        vdl = [vd, vd - NH, vd, vd - NH]
        masks = [(~sa) & (~sb), (~sa) & sb, sa & (~sb), sa & sb]
        new = []
        for g in range(G):
            ov = offs[g]
            mi = masks[g].astype(jnp.int32)
            pos = plsc.cumsum(mi)
            idx = (g * CAP - 1) + ov + pos
            plsc.store_scatter(ob_src, [idx], vsl[g], mask=masks[g])
            plsc.store_scatter(ob_dst, [idx], vdl[g], mask=masks[g])
            new.append(ov + plsc.all_reduce_population_count(masks[g]))
        return tuple(new)

    offs = lax.fori_loop(0, SLICE // 16, step, (zv, zv, zv, zv))

    zpad = jnp.zeros((16,), jnp.int32)
    tpad = jnp.full((16,), TRASH_H, jnp.int32)
    for g in range(G):
        ov = offs[g]
        for t in range(4):
            idx = (g * CAP + t * 16) + ov + iota
            plsc.store_scatter(ob_src, [idx], zpad)
            plsc.store_scatter(ob_dst, [idx], tpad)
        pv = ((ov + 63) >> 6) << 6
        cntv[pl.ds(g * 16, 16)] = pv
    pltpu.sync_copy(cntv, counts_hbm.at[wid])
    pltpu.sync_copy(ob_src, lsrc_hbm.at[wid])
    pltpu.sync_copy(ob_dst, ldst_hbm.at[wid])


_filter = pl.kernel(
    _filter_body,
    out_type=(
        jax.ShapeDtypeStruct((FT, G * CAP), jnp.int32),
        jax.ShapeDtypeStruct((FT, G * CAP), jnp.int32),
        jax.ShapeDtypeStruct((FT, G * 16), jnp.int32),
    ),
    mesh=_mesh,
    compiler_params=_sc_params,
    scratch_types=[
        pltpu.VMEM((SLICE,), jnp.int32),
        pltpu.VMEM((SLICE,), jnp.int32),
        pltpu.VMEM((G * CAP,), jnp.int32),
        pltpu.VMEM((G * CAP,), jnp.int32),
        pltpu.VMEM((G * 16,), jnp.int32),
    ],
)


def _make_pass(pass_idx):
    def body(m_hbm, lsrc, ldst, counts_hbm, *rest):
        (out_hbm, m_s, agg_s, counts_v, sv, dv, r0, r1, s0, s1) = rest
        cid = lax.axis_index("c")
        sid = lax.axis_index("s")
        a = cid
        b = cid if pass_idx == 0 else 1 - cid
        g = a * 2 + b
        rows = [r0, r1]
        ss = [s0, s1]

        pl.when(sid < 15)(lambda: pltpu.sync_copy(
            m_hbm.at[pl.ds(a * NH + sid * SROW, SROW)],
            m_s.at[pl.ds(sid * SROW, SROW)]))
        pl.when(sid == 15)(lambda: pltpu.sync_copy(
            m_hbm.at[pl.ds(a * NH + 15 * SROW, NH - 15 * SROW)],
            m_s.at[pl.ds(15 * SROW, NH - 15 * SROW)]))

        zv = jnp.zeros((16,), jnp.float32)

        def _zrow(r, carry):
            for jj in range(8):
                r0[r, pl.ds(jj * 16, 16)] = zv
            return carry

        lax.fori_loop(0, CH, _zrow, 0)
        for k in range(SROW // CH):
            pltpu.sync_copy(
                r0, agg_s.at[pl.ds(sid * SROW + k * CH, CH)])
        pltpu.sync_copy(counts_hbm, counts_v)
        plsc.subcore_barrier()

        def sstart(s, bb):
            pltpu.async_copy(rows[bb], agg_s.at[dv.at[s]], ss[bb], add=True)

        def swait(s, bb):
            pltpu.make_async_copy(rows[bb], agg_s.at[dv.at[s]],
                                  ss[bb]).wait()

        for j in range(2):
            ft = sid * 2 + j
            n = counts_v[ft, pl.ds(g * 16, 16)][0]
            nch = n >> 6
            nb = (nch + 31) >> 5

            def blk_body(bi, carry, ft=ft):
                pltpu.sync_copy(lsrc.at[ft, g, bi], sv)
                pltpu.sync_copy(ldst.at[ft, g, bi], dv)
                base = bi * 32
                for s in range(32):
                    bb = s % 2

                    def do(s=s, bb=bb):
                        if s >= 2:
                            swait(s - 2, bb)
                        pltpu.sync_copy(
                            m_s.at[sv.at[pl.ds(s * CH, CH)]], rows[bb])
                        sstart(s, bb)

                    pl.when(base + s < nch)(do)
                for s in range(32):
                    started = base + s < nch
                    last2 = (s >= 30) | (base + s >= nch - 2)
                    pl.when(started & last2)(lambda s=s: swait(s, s % 2))
                return carry

            lax.fori_loop(0, nb, blk_body, 0)

        plsc.subcore_barrier()
        pltpu.sync_copy(agg_s.at[pl.ds(sid * SROW, SROW)],
                        out_hbm.at[b, pl.ds(sid * SROW, SROW)])

    return body


def _pass_scratch():
    return [
        pltpu.VMEM_SHARED((LH, D), jnp.float32),
        pltpu.VMEM_SHARED((LH, D), jnp.float32),
        pltpu.VMEM((FT, G * 16), jnp.int32),
        pltpu.VMEM((BLK,), jnp.int32),
        pltpu.VMEM((32, CH), jnp.int32),
        pltpu.VMEM((CH, D), jnp.float32),
        pltpu.VMEM((CH, D), jnp.float32),
        pltpu.SemaphoreType.DMA,
        pltpu.SemaphoreType.DMA,
    ]


_pass0 = pl.kernel(
    _make_pass(0),
    out_type=jax.ShapeDtypeStruct((NC, LH, D), jnp.float32),
    mesh=_mesh,
    compiler_params=_sc_params,
    scratch_types=_pass_scratch(),
)

_pass1 = pl.kernel(
    _make_pass(1),
    out_type=jax.ShapeDtypeStruct((NC, LH, D), jnp.float32),
    mesh=_mesh,
    compiler_params=_sc_params,
    scratch_types=_pass_scratch(),
)


def _dot(a, b):
    return jnp.dot(a, b, precision=lax.Precision.HIGHEST,
                   preferred_element_type=jnp.float32)


def _lin_in_body(x_ref, wT_ref, b_ref, cw_ref, h_ref, m_ref):
    h = _dot(x_ref[...], wT_ref[...]) + b_ref[...]
    h_ref[...] = h
    m_ref[...] = _dot(h, cw_ref[...])


_lin_in = pl.pallas_call(
    _lin_in_body,
    grid=(N // BR,),
    in_specs=[
        pl.BlockSpec((BR, D), lambda i: (i, 0)),
        pl.BlockSpec((D, D), lambda i: (0, 0)),
        pl.BlockSpec((1, D), lambda i: (0, 0)),
        pl.BlockSpec((D, D), lambda i: (0, 0)),
    ],
    out_specs=[
        pl.BlockSpec((BR, D), lambda i: (i, 0)),
        pl.BlockSpec((BR, D), lambda i: (i, 0)),
    ],
    out_shape=[jax.ShapeDtypeStruct((N, D), jnp.float32)] * 2,
)


def _gru_body(agg_ref, agg2_ref, h_ref, wihT_ref, whhT_ref, bih_ref,
              bhh_ref, nw_ref, nb_ref, ho_ref, y_ref):
    agg = agg_ref[0] + agg2_ref[0]
    h = h_ref[...]
    gi = _dot(agg, wihT_ref[...]) + bih_ref[...]
    gh = _dot(h, whhT_ref[...]) + bhh_ref[...]
    r = jax.nn.sigmoid(gi[:, :D] + gh[:, :D])
    z = jax.nn.sigmoid(gi[:, D:2 * D] + gh[:, D:2 * D])
    n = jnp.tanh(gi[:, 2 * D:] + r * gh[:, 2 * D:])
    hn = (1.0 - z) * n + z * h
    ho_ref[...] = hn
    y_ref[...] = _dot(hn, nw_ref[...]) + nb_ref[...]


_gru = pl.pallas_call(
    _gru_body,
    grid=(N // BR,),
    in_specs=[
        pl.BlockSpec((1, BR, D), lambda i: (i // 5, i % 5, 0)),
        pl.BlockSpec((1, BR, D), lambda i: (i // 5, i % 5, 0)),
        pl.BlockSpec((BR, D), lambda i: (i, 0)),
        pl.BlockSpec((D, 3 * D), lambda i: (0, 0)),
        pl.BlockSpec((D, 3 * D), lambda i: (0, 0)),
        pl.BlockSpec((1, 3 * D), lambda i: (0, 0)),
        pl.BlockSpec((1, 3 * D), lambda i: (0, 0)),
        pl.BlockSpec((D, D), lambda i: (0, 0)),
        pl.BlockSpec((1, D), lambda i: (0, 0)),
    ],
    out_specs=[
        pl.BlockSpec((BR, D), lambda i: (i, 0)),
        pl.BlockSpec((BR, D), lambda i: (i, 0)),
    ],
    out_shape=[jax.ShapeDtypeStruct((N, D), jnp.float32)] * 2,
)


def kernel(x, edge_index, W_in, b_in, conv_w, gru_w_ih, gru_w_hh,
           gru_b_ih, gru_b_hh, W_out, b_out):
    src = edge_index[0]
    dst = edge_index[1]
    pad = E_PAD - E
    src_p = jnp.concatenate([src, jnp.zeros((pad,), src.dtype)])
    dst_p = jnp.concatenate([dst, jnp.full((pad,), N, dst.dtype)])

    lsrc, ldst, counts = _filter(src_p, dst_p)
    lsrc = lsrc.reshape(FT, G, NBLK, BLK)
    ldst = ldst.reshape(FT, G, NBLK, 32, CH)

    h, m = _lin_in(x, W_in.T, b_in.reshape(1, D), conv_w[0])

    w_ihT = gru_w_ih.T
    w_hhT = gru_w_hh.T
    b_ih2 = gru_b_ih.reshape(1, 3 * D)
    b_hh2 = gru_b_hh.reshape(1, 3 * D)
    zero_b = jnp.zeros((1, D), jnp.float32)
    nexts = [(conv_w[1], zero_b), (conv_w[2], zero_b),
             (W_out.T, b_out.reshape(1, D))]
    for i in range(L):
        p0 = _pass0(m, lsrc, ldst, counts)
        p1 = _pass1(m, lsrc, ldst, counts)
        h, m = _gru(p0, p1, h, w_ihT, w_hhT, b_ih2, b_hh2,
                    nexts[i][0], nexts[i][1])
    return m


# speedup vs baseline: 1.1493x; 1.1482x over previous
import functools

import jax
import jax.numpy as jnp
from jax import lax
from jax.experimental import pallas as pl
from jax.experimental.pallas import tpu as pltpu
from jax.experimental.pallas import tpu_sc as plsc

N = 10000
D = 128
L = 3
E = 320000

NC = 2
NS = 16
FT = NC * NS
NH = N // 2
LH = 5120
TRASH_H = NH
SLICE = 10240
E_PAD = FT * SLICE
G = 4
CAP = 12288
BLK = 2048
NBLK = CAP // BLK
CH = 64
SROW = 320
BR = 1000

_mesh = plsc.VectorSubcoreMesh(core_axis_name="c", subcore_axis_name="s")
_sc_params = pltpu.CompilerParams(needs_layout_passes=False)


def _filter_body(src_hbm, dst_hbm, lsrc_hbm, ldst_hbm, counts_hbm,
                 sv, dv, ob_src, ob_dst, cntv):
    cid = lax.axis_index("c")
    sid = lax.axis_index("s")
    wid = cid * NS + sid
    pltpu.sync_copy(src_hbm.at[pl.ds(wid * SLICE, SLICE)], sv)
    pltpu.sync_copy(dst_hbm.at[pl.ds(wid * SLICE, SLICE)], dv)

    zv = jnp.zeros((16,), jnp.int32)
    iota = lax.iota(jnp.int32, 16)

    def step(k, offs):
        vs = sv[pl.ds(k * 16, 16)]
        vd = dv[pl.ds(k * 16, 16)]
        sa = vs >= NH
        sb = vd >= NH
        vsl = [vs, vs, vs - NH, vs - NH]
        vdl = [vd, vd - NH, vd, vd - NH]
        masks = [(~sa) & (~sb), (~sa) & sb, sa & (~sb), sa & sb]
        new = []
        for g in range(G):
            ov = offs[g]
            mi = masks[g].astype(jnp.int32)
            pos = plsc.cumsum(mi)
            idx = (g * CAP - 1) + ov + pos
            plsc.store_scatter(ob_src, [idx], vsl[g], mask=masks[g])
            plsc.store_scatter(ob_dst, [idx], vdl[g], mask=masks[g])
            new.append(ov + plsc.all_reduce_population_count(masks[g]))
        return tuple(new)

    offs = lax.fori_loop(0, SLICE // 16, step, (zv, zv, zv, zv))

    zpad = jnp.zeros((16,), jnp.int32)
    tpad = jnp.full((16,), TRASH_H, jnp.int32)
    for g in range(G):
        ov = offs[g]
        for t in range(4):
            idx = (g * CAP + t * 16) + ov + iota
            plsc.store_scatter(ob_src, [idx], zpad)
            plsc.store_scatter(ob_dst, [idx], tpad)
        pv = ((ov + 63) >> 6) << 6
        cntv[pl.ds(g * 16, 16)] = pv
    pltpu.sync_copy(cntv, counts_hbm.at[wid])
    pltpu.sync_copy(ob_src, lsrc_hbm.at[wid])
    pltpu.sync_copy(ob_dst, ldst_hbm.at[wid])


_filter = pl.kernel(
    _filter_body,
    out_type=(
        jax.ShapeDtypeStruct((FT, G * CAP), jnp.int32),
        jax.ShapeDtypeStruct((FT, G * CAP), jnp.int32),
        jax.ShapeDtypeStruct((FT, G * 16), jnp.int32),
    ),
    mesh=_mesh,
    compiler_params=_sc_params,
    scratch_types=[
        pltpu.VMEM((SLICE,), jnp.int32),
        pltpu.VMEM((SLICE,), jnp.int32),
        pltpu.VMEM((G * CAP,), jnp.int32),
        pltpu.VMEM((G * CAP,), jnp.int32),
        pltpu.VMEM((G * 16,), jnp.int32),
    ],
)


def _make_pass(pass_idx):
    def body(m_hbm, lsrc, ldst, counts_hbm, *rest):
        (out_hbm, m_s, agg_s, counts_v, sv, dv, r0, r1, s0, s1) = rest
        cid = lax.axis_index("c")
        sid = lax.axis_index("s")
        a = cid
        b = cid if pass_idx == 0 else 1 - cid
        g = a * 2 + b
        rows = [r0, r1]
        ss = [s0, s1]

        pl.when(sid < 15)(lambda: pltpu.sync_copy(
            m_hbm.at[pl.ds(a * NH + sid * SROW, SROW)],
            m_s.at[pl.ds(sid * SROW, SROW)]))
        pl.when(sid == 15)(lambda: pltpu.sync_copy(
            m_hbm.at[pl.ds(a * NH + 15 * SROW, NH - 15 * SROW)],
            m_s.at[pl.ds(15 * SROW, NH - 15 * SROW)]))

        zv = jnp.zeros((16,), jnp.float32)

        def _zrow(r, carry):
            for jj in range(8):
                r0[r, pl.ds(jj * 16, 16)] = zv
            return carry

        lax.fori_loop(0, CH, _zrow, 0)
        for k in range(SROW // CH):
            pltpu.sync_copy(
                r0, agg_s.at[pl.ds(sid * SROW + k * CH, CH)])
        pltpu.sync_copy(counts_hbm, counts_v)
        plsc.subcore_barrier()

        def sstart(s, bb):
            pltpu.async_copy(rows[bb], agg_s.at[dv.at[s]], ss[bb], add=True)

        def swait(s, bb):
            pltpu.make_async_copy(rows[bb], agg_s.at[dv.at[s]],
                                  ss[bb]).wait()

        for j in range(2):
            ft = sid * 2 + j
            n = counts_v[ft, pl.ds(g * 16, 16)][0]
            nch = n >> 6
            nb = (nch + 31) >> 5

            def blk_body(bi, carry, ft=ft):
                pltpu.sync_copy(lsrc.at[ft, g, bi], sv)
                pltpu.sync_copy(ldst.at[ft, g, bi], dv)
                base = bi * 32
                for s in range(32):
                    bb = s % 2

                    def do(s=s, bb=bb):
                        if s >= 2:
                            swait(s - 2, bb)
                        pltpu.sync_copy(
                            m_s.at[sv.at[pl.ds(s * CH, CH)]], rows[bb])
                        sstart(s, bb)

                    pl.when(base + s < nch)(do)
                for s in range(32):
                    started = base + s < nch
                    last2 = (s >= 30) | (base + s >= nch - 2)
                    pl.when(started & last2)(lambda s=s: swait(s, s % 2))
                return carry

            lax.fori_loop(0, nb, blk_body, 0)

        plsc.subcore_barrier()
        pltpu.sync_copy(agg_s.at[pl.ds(sid * SROW, SROW)],
                        out_hbm.at[b, pl.ds(sid * SROW, SROW)])

    return body


def _pass_scratch():
    return [
        pltpu.VMEM_SHARED((LH, D), jnp.float32),
        pltpu.VMEM_SHARED((LH, D), jnp.float32),
        pltpu.VMEM((FT, G * 16), jnp.int32),
        pltpu.VMEM((BLK,), jnp.int32),
        pltpu.VMEM((32, CH), jnp.int32),
        pltpu.VMEM((CH, D), jnp.float32),
        pltpu.VMEM((CH, D), jnp.float32),
        pltpu.SemaphoreType.DMA,
        pltpu.SemaphoreType.DMA,
    ]


_pass0 = pl.kernel(
    _make_pass(0),
    out_type=jax.ShapeDtypeStruct((NC, LH, D), jnp.float32),
    mesh=_mesh,
    compiler_params=_sc_params,
    scratch_types=_pass_scratch(),
)

_pass1 = pl.kernel(
    _make_pass(1),
    out_type=jax.ShapeDtypeStruct((NC, LH, D), jnp.float32),
    mesh=_mesh,
    compiler_params=_sc_params,
    scratch_types=_pass_scratch(),
)


def _dot(a, b):
    return jnp.dot(a, b, precision=lax.Precision.HIGHEST,
                   preferred_element_type=jnp.float32)


def _lin_in_body(x_ref, wT_ref, b_ref, cw_ref, h_ref, m_ref):
    h = _dot(x_ref[...], wT_ref[...]) + b_ref[...]
    h_ref[...] = h
    m_ref[...] = _dot(h, cw_ref[...])


_lin_in = pl.pallas_call(
    _lin_in_body,
    grid=(N // BR,),
    in_specs=[
        pl.BlockSpec((BR, D), lambda i: (i, 0)),
        pl.BlockSpec((D, D), lambda i: (0, 0)),
        pl.BlockSpec((1, D), lambda i: (0, 0)),
        pl.BlockSpec((D, D), lambda i: (0, 0)),
    ],
    out_specs=[
        pl.BlockSpec((BR, D), lambda i: (i, 0)),
        pl.BlockSpec((BR, D), lambda i: (i, 0)),
    ],
    out_shape=[jax.ShapeDtypeStruct((N, D), jnp.float32)] * 2,
)


def _gru_body(agg_ref, agg2_ref, h_ref, wihT_ref, whhT_ref, bih_ref,
              bhh_ref, nw_ref, nb_ref, ho_ref, y_ref):
    agg = agg_ref[0] + agg2_ref[0]
    h = h_ref[...]
    gi = _dot(agg, wihT_ref[...]) + bih_ref[...]
    gh = _dot(h, whhT_ref[...]) + bhh_ref[...]
    r = jax.nn.sigmoid(gi[:, :D] + gh[:, :D])
    z = jax.nn.sigmoid(gi[:, D:2 * D] + gh[:, D:2 * D])
    n = jnp.tanh(gi[:, 2 * D:] + r * gh[:, 2 * D:])
    hn = (1.0 - z) * n + z * h
    ho_ref[...] = hn
    y_ref[...] = _dot(hn, nw_ref[...]) + nb_ref[...]


_gru = pl.pallas_call(
    _gru_body,
    grid=(N // BR,),
    in_specs=[
        pl.BlockSpec((1, BR, D), lambda i: (i // 5, i % 5, 0)),
        pl.BlockSpec((1, BR, D), lambda i: (i // 5, i % 5, 0)),
        pl.BlockSpec((BR, D), lambda i: (i, 0)),
        pl.BlockSpec((D, 3 * D), lambda i: (0, 0)),
        pl.BlockSpec((D, 3 * D), lambda i: (0, 0)),
        pl.BlockSpec((1, 3 * D), lambda i: (0, 0)),
        pl.BlockSpec((1, 3 * D), lambda i: (0, 0)),
        pl.BlockSpec((D, D), lambda i: (0, 0)),
        pl.BlockSpec((1, D), lambda i: (0, 0)),
    ],
    out_specs=[
        pl.BlockSpec((BR, D), lambda i: (i, 0)),
        pl.BlockSpec((BR, D), lambda i: (i, 0)),
    ],
    out_shape=[jax.ShapeDtypeStruct((N, D), jnp.float32)] * 2,
)


def kernel(x, edge_index, W_in, b_in, conv_w, gru_w_ih, gru_w_hh,
           gru_b_ih, gru_b_hh, W_out, b_out):
    src = edge_index[0]
    dst = edge_index[1]
    epw = E // FT
    ppw = SLICE - epw
    pad_src = jnp.zeros((FT, ppw), src.dtype)
    pad_dst = jnp.broadcast_to(
        N + (jnp.arange(ppw, dtype=dst.dtype) % (LH - NH)), (FT, ppw))
    src_p = jnp.concatenate([src.reshape(FT, epw), pad_src], 1).reshape(-1)
    dst_p = jnp.concatenate([dst.reshape(FT, epw), pad_dst], 1).reshape(-1)

    lsrc, ldst, counts = _filter(src_p, dst_p)
    lsrc = lsrc.reshape(FT, G, NBLK, BLK)
    ldst = ldst.reshape(FT, G, NBLK, 32, CH)

    h, m = _lin_in(x, W_in.T, b_in.reshape(1, D), conv_w[0])

    w_ihT = gru_w_ih.T
    w_hhT = gru_w_hh.T
    b_ih2 = gru_b_ih.reshape(1, 3 * D)
    b_hh2 = gru_b_hh.reshape(1, 3 * D)
    zero_b = jnp.zeros((1, D), jnp.float32)
    nexts = [(conv_w[1], zero_b), (conv_w[2], zero_b),
             (W_out.T, b_out.reshape(1, D))]
    for i in range(L):
        p0 = _pass0(m, lsrc, ldst, counts)
        p1 = _pass1(m, lsrc, ldst, counts)
        h, m = _gru(p0, p1, h, w_ihT, w_hhT, b_ih2, b_hh2,
                    nexts[i][0], nexts[i][1])
    return m
